# baseline (speedup 1.0000x reference)
import functools

import jax
import jax.numpy as jnp
import numpy as np
from jax import lax
from jax.experimental import pallas as pl
from jax.experimental.pallas import tpu as pltpu
from jax.experimental.pallas import tpu_sc as plsc

_D = 256
_H = 8
_PTS = 12
_LQ = 300
_B = 8
_LEVELS = ((80, 80), (40, 40), (20, 20))
_SEQ = sum(h * w for h, w in _LEVELS)
_HD = _D // _H
_BQ = _B * _LQ

_j = np.arange(_H * _PTS)
_p_of = _j % _PTS
_lvl = _p_of // 4
_Wf = np.array([w for _, w in _LEVELS], np.float32)[_lvl][None]
_Hf = np.array([h for h, _ in _LEVELS], np.float32)[_lvl][None]
_Wi = _Wf.astype(np.int32)
_BASEi = np.array([0, 6400, 8000], np.int32)[_lvl][None]
_HEADi = (_j // _PTS).astype(np.int32)[None]
_HSUM = ((_j[:, None] // _PTS) == (_j[None, :] // _PTS)).astype(np.float32)


def _tc_body(q_ref, rp_ref, wox_ref, woy_ref, wa_ref, box_ref, boy_ref,
             ba_ref, wf_ref, hf_ref, wi_ref, base_ref, head_ref, hsum_ref,
             idx_ref, wts_ref):
    b = pl.program_id(0)
    q = q_ref[0]
    sx = jnp.dot(q, wox_ref[...], preferred_element_type=jnp.float32) + box_ref[...]
    sy = jnp.dot(q, woy_ref[...], preferred_element_type=jnp.float32) + boy_ref[...]
    a = jnp.dot(q, wa_ref[...], preferred_element_type=jnp.float32) + ba_ref[...]
    e = jnp.exp(a)
    aw = e / jnp.dot(e, hsum_ref[...], preferred_element_type=jnp.float32)

    rp = rp_ref[0]
    wf = wf_ref[...]
    hf = hf_ref[...]
    locx = rp[:, 0:1] + sx * (rp[:, 2:3] * 0.125)
    locy = rp[:, 1:2] + sy * (rp[:, 3:4] * 0.125)
    ix = locx * wf - 0.5
    iy = locy * hf - 0.5
    ix0 = jnp.floor(ix)
    iy0 = jnp.floor(iy)
    fx = ix - ix0
    fy = iy - iy0
    ix1 = ix0 + 1.0
    iy1 = iy0 + 1.0
    vx0 = ((ix0 >= 0.0) & (ix0 < wf)).astype(jnp.float32)
    vx1 = ((ix1 >= 0.0) & (ix1 < wf)).astype(jnp.float32)
    vy0 = ((iy0 >= 0.0) & (iy0 < hf)).astype(jnp.float32)
    vy1 = ((iy1 >= 0.0) & (iy1 < hf)).astype(jnp.float32)
    wx = ((1.0 - fx) * vx0, fx * vx1)
    wy = ((1.0 - fy) * vy0, fy * vy1)
    cx = (jnp.clip(ix0, 0.0, wf - 1.0).astype(jnp.int32),
          jnp.clip(ix1, 0.0, wf - 1.0).astype(jnp.int32))
    cy = (jnp.clip(iy0, 0.0, hf - 1.0).astype(jnp.int32),
          jnp.clip(iy1, 0.0, hf - 1.0).astype(jnp.int32))
    wi = wi_ref[...]
    rbase = base_ref[...] + (b * _SEQ)
    head = head_ref[...]
    for c in range(4):
        jy, jx = c // 2, c % 2
        s = rbase + cy[jy] * wi + cx[jx]
        idx_ref[c, 0] = s * _H + head
        wts_ref[c, 0] = wy[jy] * wx[jx] * aw


def _tc_idx_wts(query, reference_points, W_off, b_off, W_attn, b_attn):
    wox = W_off[:, 0::2]
    woy = W_off[:, 1::2]
    box = b_off[0::2][None]
    boy = b_off[1::2][None]
    ba = b_attn[None]
    kd = pl.BlockSpec((1, _LQ, _D), lambda b: (b, 0, 0))
    kr = pl.BlockSpec((1, _LQ, 4), lambda b: (b, 0, 0))
    kw = pl.BlockSpec((_D, _H * _PTS), lambda b: (0, 0))
    kv = pl.BlockSpec((1, _H * _PTS), lambda b: (0, 0))
    ks = pl.BlockSpec((_H * _PTS, _H * _PTS), lambda b: (0, 0))
    ko = pl.BlockSpec((4, 1, _LQ, _H * _PTS), lambda b: (0, b, 0, 0))
    idx, wts = pl.pallas_call(
        _tc_body,
        grid=(_B,),
        in_specs=[kd, kr, kw, kw, kw, kv, kv, kv, kv, kv, kv, kv, kv, ks],
        out_specs=[ko, ko],
        out_shape=[
            jax.ShapeDtypeStruct((4, _B, _LQ, _H * _PTS), jnp.int32),
            jax.ShapeDtypeStruct((4, _B, _LQ, _H * _PTS), jnp.float32),
        ],
    )(query, reference_points, wox, woy, W_attn, box, boy, ba,
      jnp.asarray(_Wf), jnp.asarray(_Hf), jnp.asarray(_Wi),
      jnp.asarray(_BASEi), jnp.asarray(_HEADi), jnp.asarray(_HSUM))
    return idx, wts


def _make_sc_gather(bq_pad):
    info = plsc.get_sparse_core_info()
    nw = info.num_cores * info.num_subcores
    spw = bq_pad // nw
    hp = _H * _PTS

    @functools.partial(
        pl.kernel,
        mesh=plsc.VectorSubcoreMesh(core_axis_name="c", subcore_axis_name="s"),
        out_type=jax.ShapeDtypeStruct((bq_pad, _H, _HD), jnp.float32),
        scratch_types=[
            pltpu.VMEM((4, spw, hp), jnp.int32),
            pltpu.VMEM((4, spw, hp), jnp.float32),
            pltpu.VMEM((2, 4, hp, _HD // 2), jnp.int32),
            pltpu.VMEM((spw, _H, _HD), jnp.float32),
            pltpu.SemaphoreType.DMA,
            pltpu.SemaphoreType.DMA,
        ],
        compiler_params=pltpu.CompilerParams(use_tc_tiling_on_sc=False,
                                             needs_layout_passes=False),
    )
    def sc_gather(table, idxh, wtsh, outh, idx_v, wts_v, g_v, out_v, sem0,
                  sem1):
        wid = lax.axis_index("s") * info.num_cores + lax.axis_index("c")
        bq0 = wid * spw
        sems = (sem0, sem1)
        for c in range(4):
            pltpu.sync_copy(idxh.at[c, pl.ds(bq0, spw)], idx_v.at[c])
            pltpu.sync_copy(wtsh.at[c, pl.ds(bq0, spw)], wts_v.at[c])

        def fire(s, buf):
            for c in range(4):
                pltpu.async_copy(table.at[idx_v.at[c, s]], g_v.at[buf, c],
                                 sems[buf])

        def drain(buf):
            for c in range(4):
                pltpu.make_async_copy(table.at[pl.ds(0, hp)],
                                      g_v.at[buf, c], sems[buf]).wait()

        def load_row(buf, c, k):
            raw = plsc.bitcast(g_v[buf, c, k, 0:16], jnp.bfloat16)
            return plsc.unpack(raw, format=plsc.PackFormat.INTERLEAVED)

        def combine(s, buf):
            wv = [[wts_v[c, s, pl.ds(16 * m, 16)] for m in range(hp // 16)]
                  for c in range(4)]
            for h in range(_H):
                accs = [jnp.zeros((16,), jnp.float32) for _ in range(8)]
                for p in range(_PTS):
                    k = h * _PTS + p
                    for c in range(4):
                        w = wv[c][k // 16][k % 16]
                        ga, gb = load_row(buf, c, k)
                        accs[2 * c] = accs[2 * c] + w * ga
                        accs[2 * c + 1] = accs[2 * c + 1] + w * gb
                out_v[s, h, 0:16] = (accs[0] + accs[2]) + (accs[4] + accs[6])
                out_v[s, h, 16:32] = (accs[1] + accs[3]) + (accs[5] + accs[7])

        fire(0, 0)

        def step2(s2, carry):
            s = 2 * s2
            fire(s + 1, 1)
            drain(0)
            combine(s, 0)
            fire(s + 2, 0)
            drain(1)
            combine(s + 1, 1)
            return carry

        lax.fori_loop(0, spw // 2, step2, 0)
        drain(0)
        combine(spw - 1, 0)
        pltpu.sync_copy(out_v, outh.at[pl.ds(bq0, spw)])

    return sc_gather


def kernel(query, reference_points, input_flatten, W_off, b_off, W_attn,
           b_attn):
    idx, wts = _tc_idx_wts(query, reference_points, W_off, b_off, W_attn,
                           b_attn)
    v16 = input_flatten.astype(jnp.bfloat16)
    table = lax.bitcast_convert_type(
        v16.reshape(_B, _SEQ, _D // 2, 2), jnp.int32
    ).reshape(_B * _SEQ * _H, _HD // 2)
    sc = _make_sc_gather(_BQ)
    out = sc(table, idx.reshape(4, _BQ, _H * _PTS),
             wts.reshape(4, _BQ, _H * _PTS))
    out = out.reshape(_BQ, _H, 2, _HD // 2)
    out = jnp.transpose(out, (0, 1, 3, 2))
    return out.reshape(_B, _LQ, _D)


# speedup vs baseline: 1.6860x; 1.6860x over previous
import functools

import jax
import jax.numpy as jnp
import numpy as np
from jax import lax
from jax.experimental import pallas as pl
from jax.experimental.pallas import tpu as pltpu
from jax.experimental.pallas import tpu_sc as plsc

_D = 256
_H = 8
_PTS = 12
_LQ = 300
_B = 8
_LEVELS = ((80, 80), (40, 40), (20, 20))
_SEQ = sum(h * w for h, w in _LEVELS)
_HD = _D // _H
_BQ = _B * _LQ

_j = np.arange(_H * _PTS)
_p_of = _j % _PTS
_lvl = _p_of // 4
_Wf = np.array([w for _, w in _LEVELS], np.float32)[_lvl][None]
_Hf = np.array([h for h, _ in _LEVELS], np.float32)[_lvl][None]
_Wi = _Wf.astype(np.int32)
_BASEi = np.array([0, 6400, 8000], np.int32)[_lvl][None]
_HEADi = (_j // _PTS).astype(np.int32)[None]
_HSUM = ((_j[:, None] // _PTS) == (_j[None, :] // _PTS)).astype(np.float32)


def _tc_body(q_ref, rp_ref, wox_ref, woy_ref, wa_ref, box_ref, boy_ref,
             ba_ref, wf_ref, hf_ref, wi_ref, base_ref, head_ref, hsum_ref,
             idx_ref, wts_ref):
    b = pl.program_id(0)
    q = q_ref[0]
    sx = jnp.dot(q, wox_ref[...], preferred_element_type=jnp.float32) + box_ref[...]
    sy = jnp.dot(q, woy_ref[...], preferred_element_type=jnp.float32) + boy_ref[...]
    a = jnp.dot(q, wa_ref[...], preferred_element_type=jnp.float32) + ba_ref[...]
    e = jnp.exp(a)
    aw = e / jnp.dot(e, hsum_ref[...], preferred_element_type=jnp.float32)

    rp = rp_ref[0]
    wf = wf_ref[...]
    hf = hf_ref[...]
    locx = rp[:, 0:1] + sx * (rp[:, 2:3] * 0.125)
    locy = rp[:, 1:2] + sy * (rp[:, 3:4] * 0.125)
    ix = locx * wf - 0.5
    iy = locy * hf - 0.5
    ix0 = jnp.floor(ix)
    iy0 = jnp.floor(iy)
    fx = ix - ix0
    fy = iy - iy0
    ix1 = ix0 + 1.0
    iy1 = iy0 + 1.0
    vx0 = ((ix0 >= 0.0) & (ix0 < wf)).astype(jnp.float32)
    vx1 = ((ix1 >= 0.0) & (ix1 < wf)).astype(jnp.float32)
    vy0 = ((iy0 >= 0.0) & (iy0 < hf)).astype(jnp.float32)
    vy1 = ((iy1 >= 0.0) & (iy1 < hf)).astype(jnp.float32)
    wx = ((1.0 - fx) * vx0, fx * vx1)
    wy = ((1.0 - fy) * vy0, fy * vy1)
    cx = (jnp.clip(ix0, 0.0, wf - 1.0).astype(jnp.int32),
          jnp.clip(ix1, 0.0, wf - 1.0).astype(jnp.int32))
    cy = (jnp.clip(iy0, 0.0, hf - 1.0).astype(jnp.int32),
          jnp.clip(iy1, 0.0, hf - 1.0).astype(jnp.int32))
    wi = wi_ref[...]
    rbase = base_ref[...] + (b * _SEQ)
    head = head_ref[...]
    for c in range(4):
        jy, jx = c // 2, c % 2
        s = rbase + cy[jy] * wi + cx[jx]
        idx_ref[c, 0] = s * _H + head
        wts_ref[c, 0] = wy[jy] * wx[jx] * aw


def _tc_idx_wts(query, reference_points, W_off, b_off, W_attn, b_attn):
    wox = W_off[:, 0::2]
    woy = W_off[:, 1::2]
    box = b_off[0::2][None]
    boy = b_off[1::2][None]
    ba = b_attn[None]
    kd = pl.BlockSpec((1, _LQ, _D), lambda b: (b, 0, 0))
    kr = pl.BlockSpec((1, _LQ, 4), lambda b: (b, 0, 0))
    kw = pl.BlockSpec((_D, _H * _PTS), lambda b: (0, 0))
    kv = pl.BlockSpec((1, _H * _PTS), lambda b: (0, 0))
    ks = pl.BlockSpec((_H * _PTS, _H * _PTS), lambda b: (0, 0))
    ko = pl.BlockSpec((4, 1, _LQ, _H * _PTS), lambda b: (0, b, 0, 0))
    idx, wts = pl.pallas_call(
        _tc_body,
        grid=(_B,),
        in_specs=[kd, kr, kw, kw, kw, kv, kv, kv, kv, kv, kv, kv, kv, ks],
        out_specs=[ko, ko],
        out_shape=[
            jax.ShapeDtypeStruct((4, _B, _LQ, _H * _PTS), jnp.int32),
            jax.ShapeDtypeStruct((4, _B, _LQ, _H * _PTS), jnp.float32),
        ],
    )(query, reference_points, wox, woy, W_attn, box, boy, ba,
      jnp.asarray(_Wf), jnp.asarray(_Hf), jnp.asarray(_Wi),
      jnp.asarray(_BASEi), jnp.asarray(_HEADi), jnp.asarray(_HSUM))
    return idx, wts


def _make_sc_gather(bq_pad):
    info = plsc.get_sparse_core_info()
    nw = info.num_cores * info.num_subcores
    spw = bq_pad // nw
    hp = _H * _PTS

    @functools.partial(
        pl.kernel,
        mesh=plsc.VectorSubcoreMesh(core_axis_name="c", subcore_axis_name="s"),
        out_type=jax.ShapeDtypeStruct((bq_pad, _H, _HD), jnp.float32),
        scratch_types=[
            pltpu.VMEM((4, spw, hp), jnp.int32),
            pltpu.VMEM((4, spw, hp), jnp.float32),
            pltpu.VMEM((2, 4, hp, _HD), jnp.float32),
            pltpu.VMEM((spw, _H, _HD), jnp.float32),
            pltpu.SemaphoreType.DMA,
            pltpu.SemaphoreType.DMA,
        ],
        compiler_params=pltpu.CompilerParams(use_tc_tiling_on_sc=False,
                                             needs_layout_passes=False),
    )
    def sc_gather(table, idxh, wtsh, outh, idx_v, wts_v, g_v, out_v, sem0,
                  sem1):
        wid = lax.axis_index("s") * info.num_cores + lax.axis_index("c")
        bq0 = wid * spw
        sems = (sem0, sem1)
        for c in range(4):
            pltpu.sync_copy(idxh.at[c, pl.ds(bq0, spw)], idx_v.at[c])
            pltpu.sync_copy(wtsh.at[c, pl.ds(bq0, spw)], wts_v.at[c])

        def fire(s, buf):
            for c in range(4):
                pltpu.async_copy(table.at[idx_v.at[c, s]], g_v.at[buf, c],
                                 sems[buf])

        def drain(buf):
            for c in range(4):
                pltpu.make_async_copy(table.at[pl.ds(0, hp)],
                                      g_v.at[buf, c], sems[buf]).wait()

        def splat(v, lane):
            idx = jnp.full((16, 1), lane, jnp.int32)
            dnums = lax.GatherDimensionNumbers(
                offset_dims=(), collapsed_slice_dims=(0,),
                start_index_map=(0,))
            return lax.gather(v, idx, dnums, (1,),
                              mode=lax.GatherScatterMode.PROMISE_IN_BOUNDS)

        def combine(s, buf):
            wv = [[wts_v[c, s, pl.ds(16 * m, 16)] for m in range(hp // 16)]
                  for c in range(4)]
            for h in range(_H):
                accs = [jnp.zeros((16,), jnp.float32) for _ in range(8)]
                for p in range(_PTS):
                    k = h * _PTS + p
                    for c in range(4):
                        w = splat(wv[c][k // 16], k % 16)
                        accs[2 * c] = accs[2 * c] + w * g_v[buf, c, k, 0:16]
                        accs[2 * c + 1] = (accs[2 * c + 1]
                                           + w * g_v[buf, c, k, 16:32])
                out_v[s, h, 0:16] = (accs[0] + accs[2]) + (accs[4] + accs[6])
                out_v[s, h, 16:32] = (accs[1] + accs[3]) + (accs[5] + accs[7])

        fire(0, 0)

        def step2(s2, carry):
            s = 2 * s2
            fire(s + 1, 1)
            drain(0)
            combine(s, 0)
            fire(s + 2, 0)
            drain(1)
            combine(s + 1, 1)
            return carry

        lax.fori_loop(0, spw // 2, step2, 0)
        drain(0)
        combine(spw - 1, 0)
        pltpu.sync_copy(out_v, outh.at[pl.ds(bq0, spw)])

    return sc_gather


def kernel(query, reference_points, input_flatten, W_off, b_off, W_attn,
           b_attn):
    idx, wts = _tc_idx_wts(query, reference_points, W_off, b_off, W_attn,
                           b_attn)
    table = input_flatten.reshape(_B * _SEQ * _H, _HD)
    sc = _make_sc_gather(_BQ)
    out = sc(table, idx.reshape(4, _BQ, _H * _PTS),
             wts.reshape(4, _BQ, _H * _PTS))
    return out.reshape(_B, _LQ, _D)


# speedup vs baseline: 1.7601x; 1.0440x over previous
import functools

import jax
import jax.numpy as jnp
import numpy as np
from jax import lax
from jax.experimental import pallas as pl
from jax.experimental.pallas import tpu as pltpu
from jax.experimental.pallas import tpu_sc as plsc

_D = 256
_H = 8
_PTS = 12
_LQ = 300
_B = 8
_LEVELS = ((80, 80), (40, 40), (20, 20))
_SEQ = sum(h * w for h, w in _LEVELS)
_HD = _D // _H
_BQ = _B * _LQ

_j = np.arange(_H * _PTS)
_p_of = _j % _PTS
_lvl = _p_of // 4
_Wf = np.array([w for _, w in _LEVELS], np.float32)[_lvl][None]
_Hf = np.array([h for h, _ in _LEVELS], np.float32)[_lvl][None]
_Wi = _Wf.astype(np.int32)
_BASEi = np.array([0, 6400, 8000], np.int32)[_lvl][None]
_HEADi = (_j // _PTS).astype(np.int32)[None]
_HSUM = ((_j[:, None] // _PTS) == (_j[None, :] // _PTS)).astype(np.float32)


def _tc_body(q_ref, rp_ref, wox_ref, woy_ref, wa_ref, box_ref, boy_ref,
             ba_ref, wf_ref, hf_ref, wi_ref, base_ref, head_ref, hsum_ref,
             idx_ref, wts_ref):
    b = pl.program_id(0)
    q = q_ref[0]
    sx = jnp.dot(q, wox_ref[...], preferred_element_type=jnp.float32) + box_ref[...]
    sy = jnp.dot(q, woy_ref[...], preferred_element_type=jnp.float32) + boy_ref[...]
    a = jnp.dot(q, wa_ref[...], preferred_element_type=jnp.float32) + ba_ref[...]
    e = jnp.exp(a)
    aw = e / jnp.dot(e, hsum_ref[...], preferred_element_type=jnp.float32)

    rp = rp_ref[0]
    wf = wf_ref[...]
    hf = hf_ref[...]
    locx = rp[:, 0:1] + sx * (rp[:, 2:3] * 0.125)
    locy = rp[:, 1:2] + sy * (rp[:, 3:4] * 0.125)
    ix = locx * wf - 0.5
    iy = locy * hf - 0.5
    ix0 = jnp.floor(ix)
    iy0 = jnp.floor(iy)
    fx = ix - ix0
    fy = iy - iy0
    ix1 = ix0 + 1.0
    iy1 = iy0 + 1.0
    vx0 = ((ix0 >= 0.0) & (ix0 < wf)).astype(jnp.float32)
    vx1 = ((ix1 >= 0.0) & (ix1 < wf)).astype(jnp.float32)
    vy0 = ((iy0 >= 0.0) & (iy0 < hf)).astype(jnp.float32)
    vy1 = ((iy1 >= 0.0) & (iy1 < hf)).astype(jnp.float32)
    wx = ((1.0 - fx) * vx0, fx * vx1)
    wy = ((1.0 - fy) * vy0, fy * vy1)
    cx = (jnp.clip(ix0, 0.0, wf - 1.0).astype(jnp.int32),
          jnp.clip(ix1, 0.0, wf - 1.0).astype(jnp.int32))
    cy = (jnp.clip(iy0, 0.0, hf - 1.0).astype(jnp.int32),
          jnp.clip(iy1, 0.0, hf - 1.0).astype(jnp.int32))
    wi = wi_ref[...]
    rbase = base_ref[...] + (b * _SEQ)
    head = head_ref[...]
    for c in range(4):
        jy, jx = c // 2, c % 2
        s = rbase + cy[jy] * wi + cx[jx]
        idx_ref[c, 0] = s * _H + head
        wts_ref[c, 0] = wy[jy] * wx[jx] * aw


def _tc_idx_wts(query, reference_points, W_off, b_off, W_attn, b_attn):
    wox = W_off[:, 0::2]
    woy = W_off[:, 1::2]
    box = b_off[0::2][None]
    boy = b_off[1::2][None]
    ba = b_attn[None]
    kd = pl.BlockSpec((1, _LQ, _D), lambda b: (b, 0, 0))
    kr = pl.BlockSpec((1, _LQ, 4), lambda b: (b, 0, 0))
    kw = pl.BlockSpec((_D, _H * _PTS), lambda b: (0, 0))
    kv = pl.BlockSpec((1, _H * _PTS), lambda b: (0, 0))
    ks = pl.BlockSpec((_H * _PTS, _H * _PTS), lambda b: (0, 0))
    ko = pl.BlockSpec((4, 1, _LQ, _H * _PTS), lambda b: (0, b, 0, 0))
    idx, wts = pl.pallas_call(
        _tc_body,
        grid=(_B,),
        in_specs=[kd, kr, kw, kw, kw, kv, kv, kv, kv, kv, kv, kv, kv, ks],
        out_specs=[ko, ko],
        out_shape=[
            jax.ShapeDtypeStruct((4, _B, _LQ, _H * _PTS), jnp.int32),
            jax.ShapeDtypeStruct((4, _B, _LQ, _H * _PTS), jnp.float32),
        ],
    )(query, reference_points, wox, woy, W_attn, box, boy, ba,
      jnp.asarray(_Wf), jnp.asarray(_Hf), jnp.asarray(_Wi),
      jnp.asarray(_BASEi), jnp.asarray(_HEADi), jnp.asarray(_HSUM))
    return idx, wts


def _make_sc_gather(bq_pad):
    info = plsc.get_sparse_core_info()
    nw = info.num_cores * info.num_subcores
    spw = bq_pad // nw
    hp = _H * _PTS

    @functools.partial(
        pl.kernel,
        mesh=plsc.VectorSubcoreMesh(core_axis_name="c", subcore_axis_name="s"),
        out_type=jax.ShapeDtypeStruct((bq_pad, _H, _HD), jnp.float32),
        scratch_types=[
            pltpu.VMEM((4, spw, hp), jnp.int32),
            pltpu.VMEM((4, spw, hp), jnp.float32),
            pltpu.VMEM((3, 4, hp, _HD), jnp.float32),
            pltpu.VMEM((spw, _H, _HD), jnp.float32),
            pltpu.SemaphoreType.DMA,
            pltpu.SemaphoreType.DMA,
            pltpu.SemaphoreType.DMA,
        ],
        compiler_params=pltpu.CompilerParams(use_tc_tiling_on_sc=False,
                                             needs_layout_passes=False),
    )
    def sc_gather(table, idxh, wtsh, outh, idx_v, wts_v, g_v, out_v, sem0,
                  sem1, sem2):
        wid = lax.axis_index("s") * info.num_cores + lax.axis_index("c")
        bq0 = wid * spw
        sems = (sem0, sem1, sem2)
        for c in range(4):
            pltpu.sync_copy(idxh.at[c, pl.ds(bq0, spw)], idx_v.at[c])
            pltpu.sync_copy(wtsh.at[c, pl.ds(bq0, spw)], wts_v.at[c])

        def fire(s, buf):
            for c in range(4):
                pltpu.async_copy(table.at[idx_v.at[c, s]], g_v.at[buf, c],
                                 sems[buf])

        def drain(buf):
            for c in range(4):
                pltpu.make_async_copy(table.at[pl.ds(0, hp)],
                                      g_v.at[buf, c], sems[buf]).wait()

        def splat(v, lane):
            idx = jnp.full((16, 1), lane, jnp.int32)
            dnums = lax.GatherDimensionNumbers(
                offset_dims=(), collapsed_slice_dims=(0,),
                start_index_map=(0,))
            return lax.gather(v, idx, dnums, (1,),
                              mode=lax.GatherScatterMode.PROMISE_IN_BOUNDS)

        def combine(s, buf):
            wv = [[wts_v[c, s, pl.ds(16 * m, 16)] for m in range(hp // 16)]
                  for c in range(4)]
            for h in range(_H):
                accs = [jnp.zeros((16,), jnp.float32) for _ in range(8)]
                for p in range(_PTS):
                    k = h * _PTS + p
                    for c in range(4):
                        w = splat(wv[c][k // 16], k % 16)
                        accs[2 * c] = accs[2 * c] + w * g_v[buf, c, k, 0:16]
                        accs[2 * c + 1] = (accs[2 * c + 1]
                                           + w * g_v[buf, c, k, 16:32])
                out_v[s, h, 0:16] = (accs[0] + accs[2]) + (accs[4] + accs[6])
                out_v[s, h, 16:32] = (accs[1] + accs[3]) + (accs[5] + accs[7])

        fire(0, 0)
        fire(1, 1)

        def step3(s3, carry):
            s = 3 * s3
            for j in range(3):
                @pl.when(s + j + 2 < spw)
                def _():
                    fire(s + j + 2, (j + 2) % 3)

                drain(j)
                combine(s + j, j)
            return carry

        lax.fori_loop(0, spw // 3, step3, 0)
        pltpu.sync_copy(out_v, outh.at[pl.ds(bq0, spw)])

    return sc_gather


def kernel(query, reference_points, input_flatten, W_off, b_off, W_attn,
           b_attn):
    idx, wts = _tc_idx_wts(query, reference_points, W_off, b_off, W_attn,
                           b_attn)
    table = input_flatten.reshape(_B * _SEQ * _H, _HD)
    sc = _make_sc_gather(_BQ)
    out = sc(table, idx.reshape(4, _BQ, _H * _PTS),
             wts.reshape(4, _BQ, _H * _PTS))
    return out.reshape(_B, _LQ, _D)


# speedup vs baseline: 1.8011x; 1.0233x over previous
import functools

import jax
import jax.numpy as jnp
import numpy as np
from jax import lax
from jax.experimental import pallas as pl
from jax.experimental.pallas import tpu as pltpu
from jax.experimental.pallas import tpu_sc as plsc

_D = 256
_H = 8
_PTS = 12
_LQ = 300
_B = 8
_LEVELS = ((80, 80), (40, 40), (20, 20))
_SEQ = sum(h * w for h, w in _LEVELS)
_HD = _D // _H
_BQ = _B * _LQ

_j = np.arange(_H * _PTS)
_p_of = _j % _PTS
_lvl = _p_of // 4
_Wf = np.array([w for _, w in _LEVELS], np.float32)[_lvl][None]
_Hf = np.array([h for h, _ in _LEVELS], np.float32)[_lvl][None]
_Wi = _Wf.astype(np.int32)
_BASEi = np.array([0, 6400, 8000], np.int32)[_lvl][None]
_HEADi = (_j // _PTS).astype(np.int32)[None]
_HSUM = ((_j[:, None] // _PTS) == (_j[None, :] // _PTS)).astype(np.float32)


def _tc_body(q_ref, rp_ref, wox_ref, woy_ref, wa_ref, box_ref, boy_ref,
             ba_ref, wf_ref, hf_ref, wi_ref, base_ref, head_ref, hsum_ref,
             idx_ref, wts_ref):
    b = pl.program_id(0)
    q = q_ref[0]
    sx = jnp.dot(q, wox_ref[...], preferred_element_type=jnp.float32) + box_ref[...]
    sy = jnp.dot(q, woy_ref[...], preferred_element_type=jnp.float32) + boy_ref[...]
    a = jnp.dot(q, wa_ref[...], preferred_element_type=jnp.float32) + ba_ref[...]
    e = jnp.exp(a)
    aw = e / jnp.dot(e, hsum_ref[...], preferred_element_type=jnp.float32)

    rp = rp_ref[0]
    wf = wf_ref[...]
    hf = hf_ref[...]
    locx = rp[:, 0:1] + sx * (rp[:, 2:3] * 0.125)
    locy = rp[:, 1:2] + sy * (rp[:, 3:4] * 0.125)
    ix = locx * wf - 0.5
    iy = locy * hf - 0.5
    ix0 = jnp.floor(ix)
    iy0 = jnp.floor(iy)
    fx = ix - ix0
    fy = iy - iy0
    ix1 = ix0 + 1.0
    iy1 = iy0 + 1.0
    vx0 = ((ix0 >= 0.0) & (ix0 < wf)).astype(jnp.float32)
    vx1 = ((ix1 >= 0.0) & (ix1 < wf)).astype(jnp.float32)
    vy0 = ((iy0 >= 0.0) & (iy0 < hf)).astype(jnp.float32)
    vy1 = ((iy1 >= 0.0) & (iy1 < hf)).astype(jnp.float32)
    wx = ((1.0 - fx) * vx0, fx * vx1)
    wy = ((1.0 - fy) * vy0, fy * vy1)
    cx = (jnp.clip(ix0, 0.0, wf - 1.0).astype(jnp.int32),
          jnp.clip(ix1, 0.0, wf - 1.0).astype(jnp.int32))
    cy = (jnp.clip(iy0, 0.0, hf - 1.0).astype(jnp.int32),
          jnp.clip(iy1, 0.0, hf - 1.0).astype(jnp.int32))
    wi = wi_ref[...]
    rbase = base_ref[...] + (b * _SEQ)
    head = head_ref[...]
    for c in range(4):
        jy, jx = c // 2, c % 2
        s = rbase + cy[jy] * wi + cx[jx]
        idx_ref[0, :, c, :] = s * _H + head
        wts_ref[c, 0] = wy[jy] * wx[jx] * aw


def _tc_idx_wts(query, reference_points, W_off, b_off, W_attn, b_attn):
    wox = W_off[:, 0::2]
    woy = W_off[:, 1::2]
    box = b_off[0::2][None]
    boy = b_off[1::2][None]
    ba = b_attn[None]
    kd = pl.BlockSpec((1, _LQ, _D), lambda b: (b, 0, 0))
    kr = pl.BlockSpec((1, _LQ, 4), lambda b: (b, 0, 0))
    kw = pl.BlockSpec((_D, _H * _PTS), lambda b: (0, 0))
    kv = pl.BlockSpec((1, _H * _PTS), lambda b: (0, 0))
    ks = pl.BlockSpec((_H * _PTS, _H * _PTS), lambda b: (0, 0))
    ko = pl.BlockSpec((4, 1, _LQ, _H * _PTS), lambda b: (0, b, 0, 0))
    ki = pl.BlockSpec((1, _LQ, 4, _H * _PTS), lambda b: (b, 0, 0, 0))
    idx, wts = pl.pallas_call(
        _tc_body,
        grid=(_B,),
        in_specs=[kd, kr, kw, kw, kw, kv, kv, kv, kv, kv, kv, kv, kv, ks],
        out_specs=[ki, ko],
        out_shape=[
            jax.ShapeDtypeStruct((_B, _LQ, 4, _H * _PTS), jnp.int32),
            jax.ShapeDtypeStruct((4, _B, _LQ, _H * _PTS), jnp.float32),
        ],
    )(query, reference_points, wox, woy, W_attn, box, boy, ba,
      jnp.asarray(_Wf), jnp.asarray(_Hf), jnp.asarray(_Wi),
      jnp.asarray(_BASEi), jnp.asarray(_HEADi), jnp.asarray(_HSUM))
    return idx, wts


def _make_sc_gather(bq_pad):
    info = plsc.get_sparse_core_info()
    nw = info.num_cores * info.num_subcores
    spw = bq_pad // nw
    hp = _H * _PTS

    @functools.partial(
        pl.kernel,
        mesh=plsc.VectorSubcoreMesh(core_axis_name="c", subcore_axis_name="s"),
        out_type=jax.ShapeDtypeStruct((bq_pad, _H, _HD), jnp.float32),
        scratch_types=[
            pltpu.VMEM((spw, 3, 128), jnp.int32),
            pltpu.VMEM((4, spw, hp), jnp.float32),
            pltpu.VMEM((3, 3, 128, _HD), jnp.float32),
            pltpu.VMEM((spw, _H, _HD), jnp.float32),
            pltpu.SemaphoreType.DMA,
            pltpu.SemaphoreType.DMA,
            pltpu.SemaphoreType.DMA,
        ],
        compiler_params=pltpu.CompilerParams(use_tc_tiling_on_sc=False,
                                             needs_layout_passes=False),
    )
    def sc_gather(table, idxh, wtsh, outh, idx_v, wts_v, g_v, out_v, sem0,
                  sem1, sem2):
        wid = lax.axis_index("s") * info.num_cores + lax.axis_index("c")
        bq0 = wid * spw
        sems = (sem0, sem1, sem2)
        pltpu.sync_copy(idxh.at[pl.ds(bq0, spw)], idx_v)
        for c in range(4):
            pltpu.sync_copy(wtsh.at[c, pl.ds(bq0, spw)], wts_v.at[c])

        def fire(s, buf):
            for j in range(3):
                pltpu.async_copy(table.at[idx_v.at[s, j]], g_v.at[buf, j],
                                 sems[buf])

        def drain(buf):
            for j in range(3):
                pltpu.make_async_copy(table.at[pl.ds(0, 128)],
                                      g_v.at[buf, j], sems[buf]).wait()

        def splat(v, lane):
            idx = jnp.full((16, 1), lane, jnp.int32)
            dnums = lax.GatherDimensionNumbers(
                offset_dims=(), collapsed_slice_dims=(0,),
                start_index_map=(0,))
            return lax.gather(v, idx, dnums, (1,),
                              mode=lax.GatherScatterMode.PROMISE_IN_BOUNDS)

        def combine(s, buf):
            wv = [[wts_v[c, s, pl.ds(16 * m, 16)] for m in range(hp // 16)]
                  for c in range(4)]
            for h in range(_H):
                accs = [jnp.zeros((16,), jnp.float32) for _ in range(8)]
                for p in range(_PTS):
                    k = h * _PTS + p
                    for c in range(4):
                        w = splat(wv[c][k // 16], k % 16)
                        r = c * hp + k
                        ga = g_v[buf, r // 128, r % 128, 0:16]
                        gb = g_v[buf, r // 128, r % 128, 16:32]
                        accs[2 * c] = accs[2 * c] + w * ga
                        accs[2 * c + 1] = accs[2 * c + 1] + w * gb
                out_v[s, h, 0:16] = (accs[0] + accs[2]) + (accs[4] + accs[6])
                out_v[s, h, 16:32] = (accs[1] + accs[3]) + (accs[5] + accs[7])

        fire(0, 0)
        fire(1, 1)

        def step3(s3, carry):
            s = 3 * s3
            for j in range(3):
                @pl.when(s + j + 2 < spw)
                def _():
                    fire(s + j + 2, (j + 2) % 3)

                drain(j)
                combine(s + j, j)
            return carry

        lax.fori_loop(0, spw // 3, step3, 0)
        pltpu.sync_copy(out_v, outh.at[pl.ds(bq0, spw)])

    return sc_gather


def kernel(query, reference_points, input_flatten, W_off, b_off, W_attn,
           b_attn):
    idx, wts = _tc_idx_wts(query, reference_points, W_off, b_off, W_attn,
                           b_attn)
    table = input_flatten.reshape(_B * _SEQ * _H, _HD)
    sc = _make_sc_gather(_BQ)
    out = sc(table, idx.reshape(_BQ, 3, 128),
             wts.reshape(4, _BQ, _H * _PTS))
    return out.reshape(_B, _LQ, _D)


# speedup vs baseline: 2.0954x; 1.1634x over previous
import functools

import jax
import jax.numpy as jnp
import numpy as np
from jax import lax
from jax.experimental import pallas as pl
from jax.experimental.pallas import tpu as pltpu
from jax.experimental.pallas import tpu_sc as plsc

_D = 256
_H = 8
_PTS = 12
_LQ = 300
_B = 8
_LEVELS = ((80, 80), (40, 40), (20, 20))
_SEQ = sum(h * w for h, w in _LEVELS)
_HD = _D // _H
_BQ = _B * _LQ

_j = np.arange(_H * _PTS)
_p_of = _j % _PTS
_lvl = _p_of // 4
_Wf = np.array([w for _, w in _LEVELS], np.float32)[_lvl][None]
_Hf = np.array([h for h, _ in _LEVELS], np.float32)[_lvl][None]
_Wi = _Wf.astype(np.int32)
_BASEi = np.array([0, 6400, 8000], np.int32)[_lvl][None]
_h = _j // _PTS
_HQi = ((_h // 4) * 32 + (_h % 4)).astype(np.int32)[None]
_HSUM = ((_j[:, None] // _PTS) == (_j[None, :] // _PTS)).astype(np.float32)


def _tc_body(q_ref, rp_ref, wox_ref, woy_ref, wa_ref, box_ref, boy_ref,
             ba_ref, wf_ref, hf_ref, wi_ref, base_ref, head_ref, hsum_ref,
             idx_ref, wts_ref):
    b = pl.program_id(0)
    q = q_ref[0]
    sx = jnp.dot(q, wox_ref[...], preferred_element_type=jnp.float32) + box_ref[...]
    sy = jnp.dot(q, woy_ref[...], preferred_element_type=jnp.float32) + boy_ref[...]
    a = jnp.dot(q, wa_ref[...], preferred_element_type=jnp.float32) + ba_ref[...]
    e = jnp.exp(a)
    aw = e / jnp.dot(e, hsum_ref[...], preferred_element_type=jnp.float32)

    rp = rp_ref[0]
    wf = wf_ref[...]
    hf = hf_ref[...]
    locx = rp[:, 0:1] + sx * (rp[:, 2:3] * 0.125)
    locy = rp[:, 1:2] + sy * (rp[:, 3:4] * 0.125)
    ix = locx * wf - 0.5
    iy = locy * hf - 0.5
    ix0 = jnp.floor(ix)
    iy0 = jnp.floor(iy)
    fx = ix - ix0
    fy = iy - iy0
    ix1 = ix0 + 1.0
    iy1 = iy0 + 1.0
    vx0 = ((ix0 >= 0.0) & (ix0 < wf)).astype(jnp.float32)
    vx1 = ((ix1 >= 0.0) & (ix1 < wf)).astype(jnp.float32)
    vy0 = ((iy0 >= 0.0) & (iy0 < hf)).astype(jnp.float32)
    vy1 = ((iy1 >= 0.0) & (iy1 < hf)).astype(jnp.float32)
    wx = ((1.0 - fx) * vx0, fx * vx1)
    wy = ((1.0 - fy) * vy0, fy * vy1)
    cx = (jnp.clip(ix0, 0.0, wf - 1.0).astype(jnp.int32),
          jnp.clip(ix1, 0.0, wf - 1.0).astype(jnp.int32))
    cy = (jnp.clip(iy0, 0.0, hf - 1.0).astype(jnp.int32),
          jnp.clip(iy1, 0.0, hf - 1.0).astype(jnp.int32))
    wi = wi_ref[...]
    rbase = base_ref[...]
    hq = head_ref[...]
    for c in range(4):
        jy, jx = c // 2, c % 2
        s = rbase + cy[jy] * wi + cx[jx]
        row = (b * (_SEQ * _H) + ((s >> 3) << 6) + ((s & 7) << 2)) + hq
        idx_ref[0, :, c, :] = row
        wts_ref[c, 0] = wy[jy] * wx[jx] * aw


def _tc_idx_wts(query, reference_points, W_off, b_off, W_attn, b_attn):
    wox = W_off[:, 0::2]
    woy = W_off[:, 1::2]
    box = b_off[0::2][None]
    boy = b_off[1::2][None]
    ba = b_attn[None]
    kd = pl.BlockSpec((1, _LQ, _D), lambda b: (b, 0, 0))
    kr = pl.BlockSpec((1, _LQ, 4), lambda b: (b, 0, 0))
    kw = pl.BlockSpec((_D, _H * _PTS), lambda b: (0, 0))
    kv = pl.BlockSpec((1, _H * _PTS), lambda b: (0, 0))
    ks = pl.BlockSpec((_H * _PTS, _H * _PTS), lambda b: (0, 0))
    ko = pl.BlockSpec((4, 1, _LQ, _H * _PTS), lambda b: (0, b, 0, 0))
    ki = pl.BlockSpec((1, _LQ, 4, _H * _PTS), lambda b: (b, 0, 0, 0))
    idx, wts = pl.pallas_call(
        _tc_body,
        grid=(_B,),
        in_specs=[kd, kr, kw, kw, kw, kv, kv, kv, kv, kv, kv, kv, kv, ks],
        out_specs=[ki, ko],
        out_shape=[
            jax.ShapeDtypeStruct((_B, _LQ, 4, _H * _PTS), jnp.int32),
            jax.ShapeDtypeStruct((4, _B, _LQ, _H * _PTS), jnp.float32),
        ],
    )(query, reference_points, wox, woy, W_attn, box, boy, ba,
      jnp.asarray(_Wf), jnp.asarray(_Hf), jnp.asarray(_Wi),
      jnp.asarray(_BASEi), jnp.asarray(_HQi), jnp.asarray(_HSUM))
    return idx, wts


def _make_sc_gather(bq_pad):
    info = plsc.get_sparse_core_info()
    nw = info.num_cores * info.num_subcores
    spw = bq_pad // nw
    hp = _H * _PTS

    @functools.partial(
        pl.kernel,
        mesh=plsc.VectorSubcoreMesh(core_axis_name="c", subcore_axis_name="s"),
        out_type=jax.ShapeDtypeStruct((bq_pad, _H, _HD), jnp.float32),
        scratch_types=[
            pltpu.VMEM((spw, 3, 128), jnp.int32),
            pltpu.VMEM((4, spw, hp), jnp.float32),
            pltpu.VMEM((3, 3, 128, _HD), jnp.float32),
            pltpu.VMEM((spw, _H, _HD), jnp.float32),
            pltpu.SemaphoreType.DMA,
            pltpu.SemaphoreType.DMA,
            pltpu.SemaphoreType.DMA,
        ],
        compiler_params=pltpu.CompilerParams(use_tc_tiling_on_sc=False,
                                             needs_layout_passes=False),
    )
    def sc_gather(table, idxh, wtsh, outh, idx_v, wts_v, g_v, out_v, sem0,
                  sem1, sem2):
        wid = lax.axis_index("s") * info.num_cores + lax.axis_index("c")
        bq0 = wid * spw
        sems = (sem0, sem1, sem2)
        pltpu.sync_copy(idxh.at[pl.ds(bq0, spw)], idx_v)
        for c in range(4):
            pltpu.sync_copy(wtsh.at[c, pl.ds(bq0, spw)], wts_v.at[c])

        def fire(s, buf):
            for j in range(3):
                pltpu.async_copy(table.at[idx_v.at[s, j]], g_v.at[buf, j],
                                 sems[buf])

        def drain(buf):
            for j in range(3):
                pltpu.make_async_copy(table.at[pl.ds(0, 128)],
                                      g_v.at[buf, j], sems[buf]).wait()

        def splat(v, lane):
            idx = jnp.full((16, 1), lane, jnp.int32)
            dnums = lax.GatherDimensionNumbers(
                offset_dims=(), collapsed_slice_dims=(0,),
                start_index_map=(0,))
            return lax.gather(v, idx, dnums, (1,),
                              mode=lax.GatherScatterMode.PROMISE_IN_BOUNDS)

        def combine(s, buf):
            wv = [[wts_v[c, s, pl.ds(16 * m, 16)] for m in range(hp // 16)]
                  for c in range(4)]
            for h in range(_H):
                accs = [jnp.zeros((16,), jnp.float32) for _ in range(8)]
                for p in range(_PTS):
                    k = h * _PTS + p
                    for c in range(4):
                        w = splat(wv[c][k // 16], k % 16)
                        r = c * hp + k
                        ga = g_v[buf, r // 128, r % 128, 0:16]
                        gb = g_v[buf, r // 128, r % 128, 16:32]
                        accs[2 * c] = accs[2 * c] + w * ga
                        accs[2 * c + 1] = accs[2 * c + 1] + w * gb
                out_v[s, h, 0:16] = (accs[0] + accs[2]) + (accs[4] + accs[6])
                out_v[s, h, 16:32] = (accs[1] + accs[3]) + (accs[5] + accs[7])

        fire(0, 0)
        fire(1, 1)

        def step3(s3, carry):
            s = 3 * s3
            for j in range(3):
                @pl.when(s + j + 2 < spw)
                def _():
                    fire(s + j + 2, (j + 2) % 3)

                drain(j)
                combine(s + j, j)
            return carry

        lax.fori_loop(0, spw // 3, step3, 0)
        pltpu.sync_copy(out_v, outh.at[pl.ds(bq0, spw)])

    return sc_gather


def kernel(query, reference_points, input_flatten, W_off, b_off, W_attn,
           b_attn):
    idx, wts = _tc_idx_wts(query, reference_points, W_off, b_off, W_attn,
                           b_attn)
    table = input_flatten.reshape(_B, _SEQ // 8, 8, 2, 128)
    table = jnp.transpose(table, (0, 1, 3, 2, 4)).reshape(_B * _SEQ * _H, _HD)
    sc = _make_sc_gather(_BQ)
    out = sc(table, idx.reshape(_BQ, 3, 128),
             wts.reshape(4, _BQ, _H * _PTS))
    return out.reshape(_B, _LQ, _D)
